# Initial kernel scaffold; baseline (speedup 1.0000x reference)
import jax
import jax.numpy as jnp
from jax.experimental import pallas as pl


def kernel(input, weights):
    raise NotImplementedError("write your pallas kernel here")


# speedup vs baseline: 1.5078x; 1.5078x over previous
import functools

import jax
import jax.numpy as jnp
from jax import lax
from jax.experimental import pallas as pl
from jax.experimental.pallas import tpu as pltpu
from jax.experimental.pallas import tpu_sc as plsc

_NUM_IDS = 16384 * 20
_D = 32
_NW = 32
_BPW = _NUM_IDS // _NW
_G = 128
_K = 16
_C = _G * _K
_NCHUNK = _BPW // _C


def _sc_gather(idx, weights):
    mesh = plsc.VectorSubcoreMesh(core_axis_name="c", subcore_axis_name="s")

    @functools.partial(
        pl.kernel,
        mesh=mesh,
        compiler_params=pltpu.CompilerParams(use_tc_tiling_on_sc=False),
        out_type=jax.ShapeDtypeStruct((_NUM_IDS, _D), jnp.float32),
        scratch_types=[
            pltpu.VMEM((_BPW // _G, _G), jnp.int32),
            pltpu.VMEM((_C, _D), jnp.float32),
            pltpu.SemaphoreType.DMA,
        ],
    )
    def k(table_hbm, idx_hbm, out_hbm, idx_v, rows_v, sem):
        wid = lax.axis_index("s") * 2 + lax.axis_index("c")
        pltpu.sync_copy(idx_hbm.at[wid], idx_v)

        for j in range(_NCHUNK):
            copies = []
            for s in range(_K):
                copies.append(pltpu.async_copy(
                    table_hbm.at[idx_v.at[j * _K + s]],
                    rows_v.at[pl.ds(s * _G, _G)],
                    sem,
                ))
            for c in copies:
                c.wait()
            pltpu.sync_copy(rows_v, out_hbm.at[pl.ds(wid * _BPW + j * _C, _C)])

    return k(weights, idx)


def kernel(input, weights):
    idx = input.reshape(_NW, _BPW // _G, _G).astype(jnp.int32)
    out = _sc_gather(idx, weights)
    return out.reshape(input.shape + (_D,))


# speedup vs baseline: 1.5104x; 1.0017x over previous
import functools

import jax
import jax.numpy as jnp
from jax import lax
from jax.experimental import pallas as pl
from jax.experimental.pallas import tpu as pltpu
from jax.experimental.pallas import tpu_sc as plsc

_NUM_IDS = 16384 * 20
_D = 32
_NW = 32
_BPW = _NUM_IDS // _NW
_G = 128
_K = 8
_C = _G * _K
_NSTEP = _BPW // _C


def _sc_gather(idx, weights):
    mesh = plsc.VectorSubcoreMesh(core_axis_name="c", subcore_axis_name="s")

    @functools.partial(
        pl.kernel,
        mesh=mesh,
        compiler_params=pltpu.CompilerParams(use_tc_tiling_on_sc=False),
        out_type=jax.ShapeDtypeStruct((_NUM_IDS, _D), jnp.float32),
        scratch_types=[
            pltpu.VMEM((_BPW // _G, _G), jnp.int32),
            pltpu.VMEM((_C, _D), jnp.float32),
            pltpu.VMEM((_C, _D), jnp.float32),
            pltpu.SemaphoreType.DMA,
            pltpu.SemaphoreType.DMA,
            pltpu.SemaphoreType.DMA,
            pltpu.SemaphoreType.DMA,
        ],
    )
    def k(table_hbm, idx_hbm, out_hbm, idx_v, rows0, rows1,
          gsem0, gsem1, psem0, psem1):
        rows = (rows0, rows1)
        gsem = (gsem0, gsem1)
        psem = (psem0, psem1)
        wid = lax.axis_index("s") * 2 + lax.axis_index("c")
        base = wid * _BPW
        pltpu.sync_copy(idx_hbm.at[wid], idx_v)

        pending_put = [None, None]
        gathers = [None, None]

        def launch_put(j):
            b = j % 2
            for c in gathers[b]:
                c.wait()
            pending_put[b] = pltpu.async_copy(
                rows[b], out_hbm.at[pl.ds(base + j * _C, _C)], psem[b])

        for j in range(_NSTEP):
            b = j % 2
            if pending_put[b] is not None:
                pending_put[b].wait()
                pending_put[b] = None
            gathers[b] = [
                pltpu.async_copy(
                    table_hbm.at[idx_v.at[j * _K + s]],
                    rows[b].at[pl.ds(s * _G, _G)],
                    gsem[b],
                )
                for s in range(_K)
            ]
            if j > 0:
                launch_put(j - 1)
        launch_put(_NSTEP - 1)
        for b in range(2):
            if pending_put[b] is not None:
                pending_put[b].wait()

    return k(weights, idx)


def kernel(input, weights):
    idx = input.reshape(_NW, _BPW // _G, _G).astype(jnp.int32)
    out = _sc_gather(idx, weights)
    return out.reshape(input.shape + (_D,))
